# baseline (speedup 1.0000x reference)
import functools

import jax
import jax.numpy as jnp
from jax import lax
from jax.experimental import pallas as pl
from jax.experimental.pallas import tpu as pltpu
from jax.experimental.pallas import tpu_sc as plsc

NC = 2
NS = 16
NW = NC * NS

B1 = 16384
HB = B1 // 2
SL = 50
D = 64
V = 1000000

KB = 128
RP = 2 * KB
CPS = HB // KB
NCHT = SL * CPS
NCH = NCHT // NW

T_COLS = 8192
T_HALF = T_COLS // 2
T_SH = 13


def _repack_body(in_ref, out_ref):
    y = in_ref[...].T
    out_ref[...] = jnp.concatenate([y[:T_HALF], y[T_HALF:]], axis=-1)


N_BLK = (V + T_COLS - 1) // T_COLS


def _table_pairs(table_t):
    return pl.pallas_call(
        _repack_body,
        out_shape=jax.ShapeDtypeStruct((N_BLK * T_HALF, 2 * D), jnp.float32),
        grid=(N_BLK,),
        in_specs=[pl.BlockSpec((D, T_COLS), lambda i: (0, i))],
        out_specs=pl.BlockSpec((T_HALF, 2 * D), lambda i: (i, 0)),
    )(table_t)


def _unpack_body(in_ref, out_ref):
    z = in_ref[0]
    out_ref[0] = jnp.concatenate([z[:, :D].T, z[:, D:].T], axis=-1)


def _unpack(out1):
    return pl.pallas_call(
        _unpack_body,
        out_shape=jax.ShapeDtypeStruct((SL, D, B1), jnp.float32),
        grid=(SL,),
        in_specs=[pl.BlockSpec((1, HB, 2 * D), lambda s: (s, 0, 0))],
        out_specs=pl.BlockSpec((1, D, B1), lambda s: (s, 0, 0)),
    )(out1.reshape(SL, HB, 2 * D))


def _sc_gather(table2, idx1d):
    mesh = plsc.VectorSubcoreMesh(
        core_axis_name="c", subcore_axis_name="s",
        num_cores=NC, num_subcores=NS)

    @functools.partial(
        pl.kernel,
        out_type=jax.ShapeDtypeStruct((B1 * SL // 2, 2 * D), jnp.float32),
        mesh=mesh,
        scratch_types=[
            pltpu.VMEM((RP,), jnp.int32),
            pltpu.VMEM((RP,), jnp.int32),
            pltpu.VMEM((RP,), jnp.int32),
            pltpu.VMEM((RP,), jnp.int32),
            pltpu.VMEM((RP,), jnp.int32),
            pltpu.VMEM((RP,), jnp.int32),
            pltpu.VMEM((RP,), jnp.int32),
            pltpu.VMEM((RP,), jnp.int32),
            pltpu.VMEM((RP, 2 * D), jnp.float32),
            pltpu.VMEM((RP, 2 * D), jnp.float32),
            pltpu.VMEM((RP, 2 * D), jnp.float32),
            pltpu.VMEM((KB, 2 * D), jnp.float32),
            pltpu.SemaphoreType.DMA,
            pltpu.SemaphoreType.DMA,
            pltpu.SemaphoreType.DMA,
            pltpu.SemaphoreType.DMA,
            pltpu.SemaphoreType.DMA,
            pltpu.SemaphoreType.DMA,
        ],
    )
    def k(table_hbm, idx_hbm, out_hbm, idx_va, idx_vb, widx_va, widx_vb,
          widx_vc, poff_va, poff_vb, poff_vc, g_va, g_vb, g_vc, out_v,
          sem_i0, sem_i1, sem_g0, sem_g1, sem_g2, sem_o):
        wid = lax.axis_index("s") * NC + lax.axis_index("c")
        cc0 = wid * NCH
        idx_v = [idx_va, idx_vb]
        widx_v = [widx_va, widx_vb, widx_vc]
        poff_v = [poff_va, poff_vb, poff_vc]
        g_v = [g_va, g_vb, g_vc]
        sem_i = [sem_i0, sem_i1]
        sem_g = [sem_g0, sem_g1, sem_g2]

        def idx_copies(ci, e):
            cc = cc0 + ci
            s = cc // CPS
            c = cc - s * CPS
            f0 = s * B1 + c * KB
            for h in range(2):
                yield pltpu.make_async_copy(
                    idx_hbm.at[pl.ds(f0 + h * HB, KB)],
                    idx_v[e].at[pl.ds(h * KB, KB)], sem_i[e])

        def vcomp(e, p):
            def body(vi, c):
                v = idx_v[e][pl.ds(vi * 16, 16)]
                widx_v[p][pl.ds(vi * 16, 16)] = (
                    lax.shift_left(lax.shift_right_logical(v, T_SH), T_SH - 1)
                    | (v & (T_HALF - 1)))
                poff_v[p][pl.ds(vi * 16, 16)] = (
                    lax.shift_right_logical(v, T_SH - 1) & 1) * D
                return c
            lax.fori_loop(0, RP // 16, body, 0)

        def fire_gather(p):
            pltpu.async_copy(table_hbm.at[widx_v[p]], g_v[p], sem_g[p])

        def wait_gather(p):
            pltpu.make_async_copy(
                table_hbm.at[widx_v[p]], g_v[p], sem_g[p]).wait()

        def select(p):
            def body(g, c):
                poff16 = poff_v[p][pl.ds(g * 16, 16)]
                for u in range(16):
                    r = g * 16 + u
                    off = poff16[u]
                    for q in range(D // 16):
                        out_v[r % KB, pl.ds((r // KB) * D + q * 16, 16)] = (
                            g_v[p][r, pl.ds(off + q * 16, 16)])
                return c
            lax.fori_loop(0, RP // 16, body, 0)

        def out_copy(ci):
            cc = cc0 + ci
            s = cc // CPS
            c = cc - s * CPS
            off = pl.multiple_of(s * HB + c * KB, 8)
            return pltpu.make_async_copy(
                out_v, out_hbm.at[pl.ds(off, KB)], sem_o)

        for d in idx_copies(0, 0):
            d.start()
        for d in idx_copies(1, 1):
            d.start()
        for d in idx_copies(0, 0):
            d.wait()
        vcomp(0, 0)
        fire_gather(0)
        for d in idx_copies(1, 1):
            d.wait()
        vcomp(1, 1)
        fire_gather(1)
        for d in idx_copies(2, 0):
            d.start()
        for d in idx_copies(3, 1):
            d.start()

        def step(ci, p, e):
            @pl.when(ci + 2 < NCH)
            def _(ci=ci, p=p, e=e):
                for d in idx_copies(ci + 2, e):
                    d.wait()
                vcomp(e, (p + 2) % 3)
                fire_gather((p + 2) % 3)

                @pl.when(ci + 4 < NCH)
                def _(ci=ci, e=e):
                    for d in idx_copies(ci + 4, e):
                        d.start()

            wait_gather(p)

            @pl.when(ci > 0)
            def _(ci=ci):
                out_copy(ci - 1).wait()

            select(p)
            out_copy(ci).start()

        def chunk6(ci6, carry):
            for b in range(6):
                step(ci6 * 6 + b, b % 3, b % 2)
            return carry

        lax.fori_loop(0, NCH // 6, chunk6, 0)
        for b in range(NCH % 6):
            ci = (NCH // 6) * 6 + b
            step(ci, ci % 3, ci % 2)
        out_copy(NCH - 1).wait()

    return k(table2, idx1d)


@jax.jit
def _lookup(embedds, input):
    table2 = _table_pairs(embedds.T)
    idx1d = input.T.reshape(-1).astype(jnp.int32)
    out1 = _sc_gather(table2, idx1d)
    return _unpack(out1).transpose(2, 0, 1)


def kernel(embedds, input):
    return _lookup(embedds, input)


# speedup vs baseline: 1.0116x; 1.0116x over previous
import functools

import jax
import jax.numpy as jnp
from jax import lax
from jax.experimental import pallas as pl
from jax.experimental.pallas import tpu as pltpu
from jax.experimental.pallas import tpu_sc as plsc

NC = 2
NS = 16
NW = NC * NS

B1 = 16384
HB = B1 // 2
SL = 50
D = 64
V = 1000000

KB = 128
RP = 2 * KB
CPS = HB // KB
NCHT = SL * CPS
NCH = NCHT // NW

T_COLS = 8192
T_HALF = T_COLS // 2
T_SH = 13


def _repack_body(in_ref, out_ref):
    y = in_ref[...].T
    out_ref[...] = jnp.concatenate([y[:T_HALF], y[T_HALF:]], axis=-1)


N_BLK = (V + T_COLS - 1) // T_COLS


def _table_pairs(table_t):
    return pl.pallas_call(
        _repack_body,
        out_shape=jax.ShapeDtypeStruct((N_BLK * T_HALF, 2 * D), jnp.float32),
        grid=(N_BLK,),
        in_specs=[pl.BlockSpec((D, T_COLS), lambda i: (0, i))],
        out_specs=pl.BlockSpec((T_HALF, 2 * D), lambda i: (i, 0)),
    )(table_t)


def _unpack_body(in_ref, out_ref):
    z = in_ref[0]
    out_ref[0] = jnp.concatenate([z[:, :D].T, z[:, D:].T], axis=-1)


def _unpack(out1):
    return pl.pallas_call(
        _unpack_body,
        out_shape=jax.ShapeDtypeStruct((SL, D, B1), jnp.float32),
        grid=(SL,),
        in_specs=[pl.BlockSpec((1, HB, 2 * D), lambda s: (s, 0, 0))],
        out_specs=pl.BlockSpec((1, D, B1), lambda s: (s, 0, 0)),
    )(out1.reshape(SL, HB, 2 * D))


def _sc_gather(table2, idx1d):
    mesh = plsc.VectorSubcoreMesh(
        core_axis_name="c", subcore_axis_name="s",
        num_cores=NC, num_subcores=NS)

    @functools.partial(
        pl.kernel,
        out_type=jax.ShapeDtypeStruct((B1 * SL // 2, 2 * D), jnp.float32),
        mesh=mesh,
        scratch_types=[
            pltpu.VMEM((RP,), jnp.int32),
            pltpu.VMEM((RP,), jnp.int32),
            pltpu.VMEM((RP,), jnp.int32),
            pltpu.VMEM((RP,), jnp.int32),
            pltpu.VMEM((RP,), jnp.int32),
            pltpu.VMEM((RP, 2 * D), jnp.float32),
            pltpu.VMEM((RP, 2 * D), jnp.float32),
            pltpu.VMEM((KB, 2 * D), jnp.float32),
            pltpu.SemaphoreType.DMA,
            pltpu.SemaphoreType.DMA,
            pltpu.SemaphoreType.DMA,
            pltpu.SemaphoreType.DMA,
        ],
    )
    def k(table_hbm, idx_hbm, out_hbm, idx_v, widx_va, widx_vb,
          poff_va, poff_vb, g_va, g_vb, out_v, sem_i, sem_g0, sem_g1,
          sem_o):
        wid = lax.axis_index("s") * NC + lax.axis_index("c")
        cc0 = wid * NCH
        widx_v = [widx_va, widx_vb]
        poff_v = [poff_va, poff_vb]
        g_v = [g_va, g_vb]
        sem_g = [sem_g0, sem_g1]

        def idx_copies(ci):
            cc = cc0 + ci
            s = cc // CPS
            c = cc - s * CPS
            f0 = s * B1 + c * KB
            for h in range(2):
                yield pltpu.make_async_copy(
                    idx_hbm.at[pl.ds(f0 + h * HB, KB)],
                    idx_v.at[pl.ds(h * KB, KB)], sem_i)

        def vcomp(p):
            def body(vi, c):
                v = idx_v[pl.ds(vi * 16, 16)]
                widx_v[p][pl.ds(vi * 16, 16)] = (
                    lax.shift_left(lax.shift_right_logical(v, T_SH), T_SH - 1)
                    | (v & (T_HALF - 1)))
                poff_v[p][pl.ds(vi * 16, 16)] = (
                    lax.shift_right_logical(v, T_SH - 1) & 1) * D
                return c
            lax.fori_loop(0, RP // 16, body, 0)

        def fire_gather(p, sem):
            pltpu.async_copy(table_hbm.at[widx_v[p]], g_v[p], sem)

        def select(p):
            def body(g, c):
                poff16 = poff_v[p][pl.ds(g * 16, 16)]
                for u in range(16):
                    r = g * 16 + u
                    off = poff16[u]
                    for q in range(D // 16):
                        out_v[r % KB, pl.ds((r // KB) * D + q * 16, 16)] = (
                            g_v[p][r, pl.ds(off + q * 16, 16)])
                return c
            lax.fori_loop(0, RP // 16, body, 0)

        def out_copy(ci):
            cc = cc0 + ci
            s = cc // CPS
            c = cc - s * CPS
            off = pl.multiple_of(s * HB + c * KB, 8)
            return pltpu.make_async_copy(
                out_v, out_hbm.at[pl.ds(off, KB)], sem_o)

        for d in idx_copies(0):
            d.start()
        for d in idx_copies(0):
            d.wait()
        vcomp(0)
        fire_gather(0, sem_g[0])
        for d in idx_copies(1):
            d.start()

        def chunk2(ci2, carry):
            for b in range(2):
                ci = ci2 * 2 + b
                p, p1 = b, 1 - b

                @pl.when(ci + 1 < NCH)
                def _(ci=ci, p1=p1):
                    for d in idx_copies(ci + 1):
                        d.wait()
                    vcomp(p1)
                    fire_gather(p1, sem_g[p1])

                    @pl.when(ci + 2 < NCH)
                    def _(ci=ci):
                        for d in idx_copies(ci + 2):
                            d.start()

                pltpu.make_async_copy(
                    table_hbm.at[widx_v[p]], g_v[p], sem_g[p]).wait()

                @pl.when(ci > 0)
                def _(ci=ci):
                    out_copy(ci - 1).wait()

                select(p)
                out_copy(ci).start()
            return carry

        lax.fori_loop(0, NCH // 2, chunk2, 0)
        out_copy(NCH - 1).wait()

    return k(table2, idx1d)


@jax.jit
def _lookup(embedds, input):
    table2 = _table_pairs(embedds.T)
    idx1d = input.T.reshape(-1).astype(jnp.int32)
    out1 = _sc_gather(table2, idx1d)
    return _unpack(out1).transpose(2, 0, 1)


def kernel(embedds, input):
    return _lookup(embedds, input)
